# baseline (speedup 1.0000x reference)
import functools

import jax
import jax.numpy as jnp
from jax import lax
from jax.experimental import pallas as pl
from jax.experimental.pallas import tpu as pltpu
from jax.experimental.pallas import tpu_sc as plsc

LANES = 16


@functools.lru_cache(maxsize=None)
def _make_sc_embed(N, D, S, V, CH, NB):
    info = plsc.get_sparse_core_info()
    NC, NS = info.num_cores, info.num_subcores
    NW = NC * NS
    B = N // S
    NQ = NW // S
    n_b = B // NQ
    J = n_b // CH
    H = D // LANES
    DH = D // 8
    W = D + 1
    R = V * S
    assert CH == 128 and D % 8 == 0 and B % (NQ * CH) == 0 and J % NB == 0

    mesh = plsc.VectorSubcoreMesh(core_axis_name="c", subcore_axis_name="s")

    @functools.partial(
        pl.kernel,
        mesh=mesh,
        out_type=jax.ShapeDtypeStruct((S, DH, B // CH, 8, CH), jnp.float32),
        scratch_types=(
            [pltpu.VMEM((V, D), jnp.float32),
             pltpu.VMEM((S, D), jnp.float32),
             pltpu.VMEM((R * W,), jnp.float32),
             pltpu.VMEM((n_b * S,), jnp.int32),
             pltpu.VMEM((n_b,), jnp.int32)]
            + [pltpu.VMEM((DH, 8, CH), jnp.float32) for _ in range(NB)]
            + [pltpu.SemaphoreType.DMA for _ in range(NB * DH + 1)]
        ),
        compiler_params=pltpu.CompilerParams(use_tc_tiling_on_sc=False,
                                             needs_layout_passes=False),
    )
    def k(tok_hbm, pos_hbm, xf_hbm, out_hbm, tok_v, pos_v, fus_v, xblk_v,
          idx_v, *rest):
        bufs = rest[:NB]
        sems = rest[NB:NB + NB * DH]
        xsem = rest[NB + NB * DH]
        wid = lax.axis_index("s") * NC + lax.axis_index("c")
        sw = wid % S
        q = wid // S
        b0 = q * n_b
        xcp = pltpu.async_copy(xf_hbm.at[pl.ds(b0 * S, n_b * S)], xblk_v, xsem)
        pltpu.sync_copy(tok_hbm, tok_v)
        pltpu.sync_copy(pos_hbm, pos_v)
        ps = [[pos_v[s, pl.ds(h * LANES, LANES)] for h in range(H)]
              for s in range(S)]
        for v in range(V):
            th = [tok_v[v, pl.ds(h * LANES, LANES)] for h in range(H)]
            for s in range(S):
                for h in range(H):
                    fus_v[pl.ds((s * V + v) * W + h * LANES, LANES)] = (
                        th[h] + ps[s][h])
        xcp.wait()
        lane = lax.broadcasted_iota(jnp.int32, (LANES,), 0)

        @pl.loop(0, n_b // (16 * LANES))
        def _(i0):
            for g16 in range(16):
                g = i0 * 16 + g16
                raw = plsc.load_gather(
                    xblk_v, [(g * LANES + lane) * S + sw])
                idx_v[pl.ds(g * LANES, LANES)] = (raw + sw * V) * W

        def fill(j, b):
            for g in range(CH // LANES):
                ga = idx_v[pl.ds(j * CH + g * LANES, LANES)]
                for d in range(D):
                    bufs[b][d // 8, d % 8, pl.ds(g * LANES, LANES)] = (
                        plsc.load_gather(fus_v, [ga + d]))

        def store(j, b):
            for dh in range(DH):
                pltpu.async_copy(bufs[b].at[dh],
                                 out_hbm.at[sw, dh, q * J + j],
                                 sems[b * DH + dh])

        def wait(b):
            for dh in range(DH):
                pltpu.make_async_copy(bufs[b].at[dh],
                                      out_hbm.at[0, 0, 0],
                                      sems[b * DH + dh]).wait()

        for b in range(NB):
            fill(b, b)
            store(b, b)

        @pl.loop(NB, J, step=NB)
        def _(j0):
            for b in range(NB):
                wait(b)
                fill(j0 + b, b)
                store(j0 + b, b)

        for b in range(NB):
            wait(b)

    return k


def kernel(x, token_table, pos_table):
    B, S = x.shape
    V, D = token_table.shape
    N = B * S
    xf = x.reshape(N)
    p = _make_sc_embed(N, D, S, V, 128, 4)(token_table, pos_table[:S], xf)
    return p.transpose(2, 4, 0, 1, 3).reshape(B, S, D)


# speedup vs baseline: 1.0795x; 1.0795x over previous
import functools

import jax
import jax.numpy as jnp
from jax import lax
from jax.experimental import pallas as pl
from jax.experimental.pallas import tpu as pltpu
from jax.experimental.pallas import tpu_sc as plsc

LANES = 16


@functools.lru_cache(maxsize=None)
def _make_sc_embed(N, D, S, V, CH, NB):
    info = plsc.get_sparse_core_info()
    NC, NS = info.num_cores, info.num_subcores
    NW = NC * NS
    B = N // S
    NQ = NW // S
    n_b = B // NQ
    J = n_b // CH
    H = D // LANES
    DH = D // 8
    W = D + 1
    R = V * S
    assert CH == 128 and D % 8 == 0 and B % (NQ * CH) == 0 and J % NB == 0

    mesh = plsc.VectorSubcoreMesh(core_axis_name="c", subcore_axis_name="s")

    @functools.partial(
        pl.kernel,
        mesh=mesh,
        out_type=jax.ShapeDtypeStruct((S, DH, B // CH, 8, CH), jnp.float32),
        scratch_types=(
            [pltpu.VMEM((V, D), jnp.float32),
             pltpu.VMEM((S, D), jnp.float32),
             pltpu.VMEM((R * W,), jnp.float32),
             pltpu.VMEM((n_b * S,), jnp.int32),
             pltpu.VMEM((n_b,), jnp.int32)]
            + [pltpu.VMEM((DH, 8, CH), jnp.float32) for _ in range(NB)]
            + [pltpu.SemaphoreType.DMA for _ in range(NB * DH + 1)]
        ),
        compiler_params=pltpu.CompilerParams(use_tc_tiling_on_sc=False,
                                             needs_layout_passes=False),
    )
    def k(tok_hbm, pos_hbm, xf_hbm, out_hbm, tok_v, pos_v, fus_v, xblk_v,
          idx_v, *rest):
        bufs = rest[:NB]
        sems = rest[NB:NB + NB * DH]
        xsem = rest[NB + NB * DH]
        wid = lax.axis_index("s") * NC + lax.axis_index("c")
        sw = wid % S
        q = wid // S
        b0 = q * n_b
        xcp = pltpu.async_copy(xf_hbm.at[pl.ds(b0 * S, n_b * S)], xblk_v, xsem)
        pltpu.sync_copy(tok_hbm, tok_v)
        pltpu.sync_copy(pos_hbm, pos_v)
        ps = [[pos_v[s, pl.ds(h * LANES, LANES)] for h in range(H)]
              for s in range(S)]
        for v in range(V):
            th = [tok_v[v, pl.ds(h * LANES, LANES)] for h in range(H)]
            for s in range(S):
                for h in range(H):
                    fus_v[pl.ds((s * V + v) * W + h * LANES, LANES)] = (
                        th[h] + ps[s][h])
        xcp.wait()
        lane = lax.broadcasted_iota(jnp.int32, (LANES,), 0)

        @pl.loop(0, n_b // (16 * LANES))
        def _(i0):
            for g16 in range(16):
                g = i0 * 16 + g16
                raw = plsc.load_gather(
                    xblk_v, [(g * LANES + lane) * S + sw])
                idx_v[pl.ds(g * LANES, LANES)] = (raw + sw * V) * W

        def fill(j, b):
            for g in range(CH // LANES):
                ga = idx_v[pl.ds(j * CH + g * LANES, LANES)]
                for d in range(D):
                    bufs[b][d // 8, d % 8, pl.ds(g * LANES, LANES)] = (
                        plsc.load_gather(fus_v, [ga + d]))

        def store(j, b):
            for dh in range(DH):
                pltpu.async_copy(bufs[b].at[dh],
                                 out_hbm.at[sw, dh, q * J + j],
                                 sems[b * DH + dh])

        def wait(b):
            for dh in range(DH):
                pltpu.make_async_copy(bufs[b].at[dh],
                                      out_hbm.at[0, 0, 0],
                                      sems[b * DH + dh]).wait()

        for b in range(NB):
            fill(b, b)
            store(b, b)

        @pl.loop(NB, J, step=NB)
        def _(j0):
            for b in range(NB):
                wait(b)
                fill(j0 + b, b)
                store(j0 + b, b)

        for b in range(NB):
            wait(b)

    return k


def kernel(x, token_table, pos_table):
    B, S = x.shape
    V, D = token_table.shape
    N = B * S
    xf = x.reshape(N)
    p = _make_sc_embed(N, D, S, V, 128, 2)(token_table, pos_table[:S], xf)
    return p.transpose(2, 4, 0, 1, 3).reshape(B, S, D)
